# Initial kernel scaffold; baseline (speedup 1.0000x reference)
import jax
import jax.numpy as jnp
from jax.experimental import pallas as pl


def kernel(features, type_table, pos_table, table_meta, db_W, db_b, db_g, db_beta, tm_W, tm_b, tm_g, tm_beta, tfin_W, tfin_b, tfin_g, tfin_beta, fin_W, fin_b, fin_g, fin_beta):
    raise NotImplementedError("write your pallas kernel here")


# speedup vs baseline: 1.8053x; 1.8053x over previous
import functools

import jax
import jax.numpy as jnp
from jax import lax
from jax.experimental import pallas as pl
from jax.experimental.pallas import tpu as pltpu
from jax.experimental.pallas import tpu_sc as plsc

N = 16384
D = 32
_NC = 2
_NS = 16
_NW = _NC * _NS
_BPW = N // _NW


def _sc_gather(type_table, pos_table, table_meta, idx_type, idx_pos, idx_tab):
    mesh = plsc.VectorSubcoreMesh(core_axis_name="c", subcore_axis_name="s")

    @functools.partial(
        pl.kernel,
        mesh=mesh,
        out_type=[jax.ShapeDtypeStruct((N, D), jnp.float32)] * 3,
        scratch_types=[
            pltpu.VMEM((_BPW,), jnp.int32),
            pltpu.VMEM((_BPW,), jnp.int32),
            pltpu.VMEM((_BPW,), jnp.int32),
            pltpu.VMEM((_BPW, D), jnp.float32),
            pltpu.VMEM((_BPW, D), jnp.float32),
            pltpu.VMEM((_BPW, D), jnp.float32),
            pltpu.SemaphoreType.DMA,
            pltpu.SemaphoreType.DMA,
            pltpu.SemaphoreType.DMA,
        ],
        compiler_params=pltpu.CompilerParams(use_tc_tiling_on_sc=False),
    )
    def k(tt, pt, tm, it_, ip_, ix_, o_t, o_p, o_m,
          iv_t, iv_p, iv_x, r_t, r_p, r_m, s_t, s_p, s_m):
        wid = lax.axis_index("s") * _NC + lax.axis_index("c")
        base = wid * _BPW
        pltpu.sync_copy(it_.at[pl.ds(base, _BPW)], iv_t)
        pltpu.sync_copy(ip_.at[pl.ds(base, _BPW)], iv_p)
        pltpu.sync_copy(ix_.at[pl.ds(base, _BPW)], iv_x)
        c1 = pltpu.async_copy(tt.at[iv_t], r_t, s_t)
        c2 = pltpu.async_copy(pt.at[iv_p], r_p, s_p)
        c3 = pltpu.async_copy(tm.at[iv_x], r_m, s_m)
        c1.wait()
        c2.wait()
        c3.wait()
        pltpu.sync_copy(r_t, o_t.at[pl.ds(base, _BPW)])
        pltpu.sync_copy(r_p, o_p.at[pl.ds(base, _BPW)])
        pltpu.sync_copy(r_m, o_m.at[pl.ds(base, _BPW)])

    return k(type_table, pos_table, table_meta, idx_type, idx_pos, idx_tab)


def _gelu(x):
    return x * 0.5 * (1.0 + lax.erf(x * 0.7071067811865476))


def _ln(x, g, b):
    m = jnp.mean(x, axis=-1, keepdims=True)
    d = x - m
    v = jnp.mean(d * d, axis=-1, keepdims=True)
    return d * lax.rsqrt(v + 1e-5) * g + b


def _dense_body(te, pe, me, db,
                dbW, dbb, dbg, dbbeta,
                tmW, tmb, tmg, tmbeta,
                tfW, tfb, tfg, tfbeta,
                fW1, fW2, fW3, fW4, fb, fg, fbeta, o):
    f32 = jnp.float32
    db_h = jnp.dot(db[...], dbW[...], preferred_element_type=f32) + dbb[...]
    db_emb = _gelu(_ln(db_h, dbg[...], dbbeta[...]))
    t = jnp.dot(me[...], tmW[...], preferred_element_type=f32) + tmb[...]
    t = _gelu(_ln(t, tmg[...], tmbeta[...]))
    t = jnp.dot(t, tfW[...], preferred_element_type=f32) + tfb[...]
    t = _gelu(_ln(t, tfg[...], tfbeta[...]))
    acc = (jnp.dot(te[...], fW1[...], preferred_element_type=f32)
           + jnp.dot(pe[...], fW2[...], preferred_element_type=f32)
           + jnp.dot(db_emb, fW3[...], preferred_element_type=f32)
           + jnp.dot(t, fW4[...], preferred_element_type=f32)
           + fb[...])
    o[...] = _gelu(_ln(acc, fg[...], fbeta[...]))


def _tc_dense(te, pe, me, db_est,
              db_W, db_b, db_g, db_beta,
              tm_W, tm_b, tm_g, tm_beta,
              tfin_W, tfin_b, tfin_g, tfin_beta,
              fin_W, fin_b, fin_g, fin_beta,
              block=2048):
    grid = (N // block,)

    def row(d):
        return pl.BlockSpec((block, d), lambda i: (i, 0))

    def full(a):
        return pl.BlockSpec(a.shape, lambda i: (0,) * a.ndim)

    fW1 = fin_W[0:32]
    fW2 = fin_W[32:64]
    fW3 = fin_W[64:96]
    fW4 = fin_W[96:160]
    vecs = [db_b, db_g, db_beta, tm_b, tm_g, tm_beta,
            tfin_b, tfin_g, tfin_beta, fin_b, fin_g, fin_beta]
    (db_b, db_g, db_beta, tm_b, tm_g, tm_beta,
     tfin_b, tfin_g, tfin_beta, fin_b, fin_g, fin_beta) = [
        v.reshape(1, -1) for v in vecs]

    args = (te, pe, me, db_est,
            db_W, db_b, db_g, db_beta,
            tm_W, tm_b, tm_g, tm_beta,
            tfin_W, tfin_b, tfin_g, tfin_beta,
            fW1, fW2, fW3, fW4, fin_b, fin_g, fin_beta)
    specs = [row(D), row(D), row(D), row(2)] + [full(a) for a in args[4:]]

    return pl.pallas_call(
        _dense_body,
        grid=grid,
        in_specs=specs,
        out_specs=pl.BlockSpec((block, 128), lambda i: (i, 0)),
        out_shape=jax.ShapeDtypeStruct((N, 128), jnp.float32),
        compiler_params=pltpu.CompilerParams(
            dimension_semantics=("parallel",)),
    )(*args)


def kernel(features, type_table, pos_table, table_meta,
           db_W, db_b, db_g, db_beta,
           tm_W, tm_b, tm_g, tm_beta,
           tfin_W, tfin_b, tfin_g, tfin_beta,
           fin_W, fin_b, fin_g, fin_beta):
    idx_type = features[:, 0].astype(jnp.int32)
    idx_pos = features[:, 1].astype(jnp.int32)
    idx_tab = features[:, 6].astype(jnp.int32)
    db_est = features[:, 2:4]
    te, pe, me = _sc_gather(type_table, pos_table, table_meta,
                            idx_type, idx_pos, idx_tab)
    return _tc_dense(te, pe, me, db_est,
                     db_W, db_b, db_g, db_beta,
                     tm_W, tm_b, tm_g, tm_beta,
                     tfin_W, tfin_b, tfin_g, tfin_beta,
                     fin_W, fin_b, fin_g, fin_beta)
